# baseline (speedup 1.0000x reference)
import functools

import jax
import jax.numpy as jnp
import numpy as np
from jax import lax
from jax.experimental import pallas as pl
from jax.experimental.pallas import tpu as pltpu
from jax.experimental.pallas import tpu_sc as plsc

N_NODES = 10000
N_EDGES = 160000
NC, NS = 2, 16
NW = NC * NS
ROW_BLK = 1000
EDGE_BLK = 2000
CH = 200
CHG = 40


def _elu(x):
    return jnp.where(x > 0, x, jnp.exp(jnp.minimum(x, 0.0)) - 1.0)


def _dot(a, b):
    return jnp.dot(a, b, preferred_element_type=jnp.float32)


def _full(a):
    return pl.BlockSpec(a.shape, lambda i: (0, 0))


def _mlp_proj_body(x_ref, wa_ref, ba_ref, wb_ref, bb_ref,
                   wq_ref, bq_ref, wk_ref, bk_ref, wv_ref, bv_ref,
                   h_ref, q_ref, k_ref, v_ref):
    h = _elu(_dot(x_ref[...], wa_ref[...]) + ba_ref[...])
    h = _elu(_dot(h, wb_ref[...]) + bb_ref[...])
    h_ref[...] = h
    q_ref[...] = _dot(h, wq_ref[...]) + bq_ref[...]
    k_ref[...] = _dot(h, wk_ref[...]) + bk_ref[...]
    v_ref[...] = _dot(h, wv_ref[...]) + bv_ref[...]


def _mlp_proj(x, wa, ba, wb, bb, wq, bq, wk, bk, wv, bv):
    n, din = x.shape
    dh = wb.shape[1]
    d = wq.shape[1]
    grid = n // ROW_BLK
    return pl.pallas_call(
        _mlp_proj_body,
        grid=(grid,),
        in_specs=[pl.BlockSpec((ROW_BLK, din), lambda i: (i, 0))]
                 + [_full(a) for a in (wa, ba, wb, bb, wq, bq, wk, bk,
                                       wv, bv)],
        out_specs=[pl.BlockSpec((ROW_BLK, dh), lambda i: (i, 0))]
                  + [pl.BlockSpec((ROW_BLK, d), lambda i: (i, 0))] * 3,
        out_shape=[jax.ShapeDtypeStruct((n, dh), jnp.float32)]
                  + [jax.ShapeDtypeStruct((n, d), jnp.float32)] * 3,
    )(x, wa, ba, wb, bb, wq, bq, wk, bk, wv, bv)


def _alpha_msg_body(qd_ref, ks_ref, vs_ref, attr_ref, we_ref, sp_ref,
                    st_ref, ea_ref, *o_refs, widths, scale):
    e = _dot(attr_ref[...], we_ref[...])
    logit = _dot(qd_ref[...] * (ks_ref[...] + e), sp_ref[...]) * scale
    ea = jnp.exp(logit)
    ea_ref[...] = ea
    m = (vs_ref[...] + e) * _dot(ea, st_ref[...])
    off = 0
    for o_ref, wd in zip(o_refs, widths):
        o_ref[...] = m[:, off:off + wd]
        off += wd


def _alpha_msg(qd, ks, vs, attr, we, sp, st, scale, widths):
    e, d = qd.shape
    grid = e // EDGE_BLK
    eb = lambda wd: pl.BlockSpec((EDGE_BLK, wd), lambda i: (i, 0))
    return pl.pallas_call(
        functools.partial(_alpha_msg_body, widths=tuple(widths),
                          scale=scale),
        grid=(grid,),
        in_specs=[eb(d), eb(d), eb(d), eb(7), _full(we), _full(sp),
                  _full(st)],
        out_specs=[eb(16)] + [eb(wd) for wd in widths],
        out_shape=[jax.ShapeDtypeStruct((e, 16), jnp.float32)]
                  + [jax.ShapeDtypeStruct((e, wd), jnp.float32)
                     for wd in widths],
    )(qd, ks, vs, attr, we, sp, st)


def _combine_block(refs, widths):
    np_ = len(widths)
    p_refs = refs[:np_]
    dp_ref, h_ref, ws_ref, bs_ref, st_ref = refs[np_:np_ + 5]
    rest = refs[np_ + 5:]
    base = _dot(h_ref[...], ws_ref[...]) + bs_ref[...]
    r = 1.0 / (dp_ref[0] + dp_ref[1] + 1e-16)
    rb = _dot(r, st_ref[...])
    cols = []
    off = 0
    for p_ref, wd in zip(p_refs, widths):
        cols.append(_elu((p_ref[0] + p_ref[1]) * rb[:, off:off + wd]
                         + base[:, off:off + wd]))
        off += wd
    o = cols[0] if len(cols) == 1 else jnp.concatenate(cols, axis=1)
    return o, rest


def _combine_proj_body(*refs, widths):
    o, rest = _combine_block(refs[:-4], widths)
    wq_ref, bq_ref, wk_ref, bk_ref, wv_ref, bv_ref = rest
    h_ref, q_ref, k_ref, v_ref = refs[-4:]
    h_ref[...] = o
    q_ref[...] = _dot(o, wq_ref[...]) + bq_ref[...]
    k_ref[...] = _dot(o, wk_ref[...]) + bk_ref[...]
    v_ref[...] = _dot(o, wv_ref[...]) + bv_ref[...]


def _combine_proj(parts, dparts, h, ws, bs, st, wq, bq, wk, bk, wv, bv):
    n, din = h.shape
    d = ws.shape[1]
    d2 = wq.shape[1]
    widths = tuple(p.shape[2] for p in parts)
    grid = n // ROW_BLK
    return pl.pallas_call(
        functools.partial(_combine_proj_body, widths=widths),
        grid=(grid,),
        in_specs=[pl.BlockSpec((2, ROW_BLK, wd), lambda i: (0, i, 0))
                  for wd in widths]
                 + [pl.BlockSpec((2, ROW_BLK, 16), lambda i: (0, i, 0)),
                    pl.BlockSpec((ROW_BLK, din), lambda i: (i, 0))]
                 + [_full(a) for a in (ws, bs, st, wq, bq, wk, bk, wv, bv)],
        out_specs=[pl.BlockSpec((ROW_BLK, d), lambda i: (i, 0))]
                  + [pl.BlockSpec((ROW_BLK, d2), lambda i: (i, 0))] * 3,
        out_shape=[jax.ShapeDtypeStruct((n, d), jnp.float32)]
                  + [jax.ShapeDtypeStruct((n, d2), jnp.float32)] * 3,
    )(*parts, dparts, h, ws, bs, st, wq, bq, wk, bk, wv, bv)


def _combine_head_body(*refs, widths):
    o, rest = _combine_block(refs[:-1], widths)
    w1_ref, b1_ref, w2_ref, b2_ref = rest
    y_ref = refs[-1]
    t = _elu(_dot(o, w1_ref[...]) + b1_ref[...])
    y_ref[...] = _dot(t, w2_ref[...]) + b2_ref[...]


def _combine_head(parts, dparts, h, ws, bs, st, w1, b1, w2, b2):
    n, din = h.shape
    widths = tuple(p.shape[2] for p in parts)
    grid = n // ROW_BLK
    return pl.pallas_call(
        functools.partial(_combine_head_body, widths=widths),
        grid=(grid,),
        in_specs=[pl.BlockSpec((2, ROW_BLK, wd), lambda i: (0, i, 0))
                  for wd in widths]
                 + [pl.BlockSpec((2, ROW_BLK, 16), lambda i: (0, i, 0)),
                    pl.BlockSpec((ROW_BLK, din), lambda i: (i, 0))]
                 + [_full(a) for a in (ws, bs, st, w1, b1, w2, b2)],
        out_specs=pl.BlockSpec((ROW_BLK, 1), lambda i: (i, 0)),
        out_shape=jax.ShapeDtypeStruct((n, 1), jnp.float32),
    )(*parts, dparts, h, ws, bs, st, w1, b1, w2, b2)


@functools.cache
def _sc_mesh():
    return plsc.VectorSubcoreMesh(core_axis_name="c", subcore_axis_name="s",
                                  num_cores=NC, num_subcores=NS)


def _gather_qkv(kt, qt, vt, src2d, dst2d):
    n_rows, d = kt.shape
    n_ch_tot, ch = src2d.shape
    e = n_ch_tot * ch
    per_w = e // NW
    n_ch = per_w // ch

    @functools.partial(
        pl.kernel, mesh=_sc_mesh(),
        out_type=[jax.ShapeDtypeStruct((e, d), jnp.float32)] * 3,
        compiler_params=pltpu.CompilerParams(use_tc_tiling_on_sc=False),
        scratch_types=[pltpu.VMEM((n_ch, ch), jnp.int32),
                       pltpu.VMEM((n_ch, ch), jnp.int32)]
                      + [pltpu.VMEM((ch, d), jnp.float32)] * 6
                      + [pltpu.SemaphoreType.DMA] * 12,
    )
    def k(kt_h, qt_h, vt_h, src_h, dst_h, ks_h, qd_h, vs_h,
          si_v, di_v, k0, k1, q0, q1, v0, v1, *sems):
        wid = lax.axis_index("s") * NC + lax.axis_index("c")
        cbase = wid * n_ch
        ebase = wid * per_w
        bufs = ((k0, k1), (q0, q1), (v0, v1))
        outs = (ks_h, qd_h, vs_h)
        tabs = (kt_h, qt_h, vt_h)
        idxs = (si_v, di_v, si_v)
        sg = (sems[0:2], sems[2:4], sems[4:6])
        sw = (sems[6:8], sems[8:10], sems[10:12])
        pltpu.sync_copy(src_h.at[pl.ds(cbase, n_ch)], si_v)
        pltpu.sync_copy(dst_h.at[pl.ds(cbase, n_ch)], di_v)

        def fire(g, b):
            for t in range(3):
                pltpu.async_copy(tabs[t].at[idxs[t].at[g]],
                                 bufs[t][b], sg[t][b])

        def wait_writes(g, b):
            for t in range(3):
                pltpu.make_async_copy(
                    bufs[t][b], outs[t].at[pl.ds(ebase + g * ch, ch)],
                    sw[t][b]).wait()

        def drain(g, b):
            for t in range(3):
                pltpu.make_async_copy(tabs[t].at[idxs[t].at[g]],
                                      bufs[t][b], sg[t][b]).wait()
                pltpu.async_copy(bufs[t][b],
                                 outs[t].at[pl.ds(ebase + g * ch, ch)],
                                 sw[t][b])

        fire(0, 0)

        def body(p, carry):
            for b in range(2):
                g = p * 2 + b
                nb = 1 - b

                @pl.when(jnp.logical_and(g + 1 < n_ch, g >= 1))
                def _():
                    wait_writes(g - 1, nb)

                @pl.when(g + 1 < n_ch)
                def _():
                    fire(g + 1, nb)

                drain(g, b)
            return carry

        lax.fori_loop(0, n_ch // 2, body, 0)
        if n_ch % 2:
            g = n_ch - 1
            drain(g, g % 2)
        wait_writes(n_ch - 2, (n_ch - 2) % 2)
        wait_writes(n_ch - 1, (n_ch - 1) % 2)

    return k(kt, qt, vt, src2d, dst2d)


def _scatter_add_multi(rows_list, idx2d, n_out):
    nr = len(rows_list)
    e = rows_list[0].shape[0]
    dims = [r.shape[1] for r in rows_list]
    n_ch_tot, ch = idx2d.shape
    per_w = e // NW
    n_ch = per_w // ch
    rpt = n_out // NS

    scratch = [pltpu.VMEM((n_ch, ch), jnp.int32)]
    for d in dims:
        scratch += [pltpu.VMEM((ch, d), jnp.float32)] * 2
    for d in dims:
        scratch.append(pltpu.VMEM_SHARED((n_out, d), jnp.float32))
    scratch += [pltpu.SemaphoreType.DMA] * (2 * nr)

    @functools.partial(
        pl.kernel, mesh=_sc_mesh(),
        out_type=[jax.ShapeDtypeStruct((NC, n_out, d), jnp.float32)
                  for d in dims],
        compiler_params=pltpu.CompilerParams(use_tc_tiling_on_sc=False),
        scratch_types=scratch,
    )
    def k(*args):
        rows_h = args[:nr]
        idx_h = args[nr]
        outs = args[nr + 1:2 * nr + 1]
        rest = args[2 * nr + 1:]
        idx_v = rest[0]
        bufs = [(rest[1 + 2 * i], rest[2 + 2 * i]) for i in range(nr)]
        accs = rest[1 + 2 * nr:1 + 3 * nr]
        sems = rest[1 + 3 * nr:]
        semp = [(sems[2 * i], sems[2 * i + 1]) for i in range(nr)]

        c = lax.axis_index("c")
        s = lax.axis_index("s")
        wid = s * NC + c
        zbase = s * rpt

        for i, d in enumerate(dims):
            b0 = bufs[i][0]

            def zero_buf(r, carry, b0=b0, d=d):
                for j in range(d // 16):
                    b0[r, pl.ds(j * 16, 16)] = jnp.zeros((16,),
                                                         jnp.float32)
                return carry

            lax.fori_loop(0, ch, zero_buf, 0)
            off = 0
            while off < rpt:
                step = min(ch, rpt - off)
                pltpu.sync_copy(b0.at[pl.ds(0, step)],
                                accs[i].at[pl.ds(zbase + off, step)])
                off += step

        cbase = wid * n_ch
        ebase = wid * per_w
        pltpu.sync_copy(idx_h.at[pl.ds(cbase, n_ch)], idx_v)
        for i in range(nr):
            pltpu.async_copy(rows_h[i].at[pl.ds(ebase, ch)],
                             bufs[i][0], semp[i][0])
        plsc.subcore_barrier()

        def body(p, carry):
            for b in range(2):
                g = p * 2 + b
                nb = 1 - b

                @pl.when(g + 1 < n_ch)
                def _():
                    for i in range(nr):
                        pltpu.async_copy(
                            rows_h[i].at[pl.ds(ebase + (g + 1) * ch, ch)],
                            bufs[i][nb], semp[i][nb])

                for i in range(nr):
                    pltpu.make_async_copy(
                        rows_h[i].at[pl.ds(ebase + g * ch, ch)],
                        bufs[i][b], semp[i][b]).wait()
                    pltpu.sync_copy(bufs[i][b], accs[i].at[idx_v.at[g]],
                                    add=True)
            return carry

        lax.fori_loop(0, n_ch // 2, body, 0)
        if n_ch % 2:
            g = n_ch - 1
            b = g % 2
            for i in range(nr):
                pltpu.make_async_copy(
                    rows_h[i].at[pl.ds(ebase + g * ch, ch)],
                    bufs[i][b], semp[i][b]).wait()
                pltpu.sync_copy(bufs[i][b], accs[i].at[idx_v.at[g]],
                                add=True)
        plsc.subcore_barrier()

        for i in range(nr):
            pltpu.sync_copy(accs[i].at[pl.ds(zbase, rpt)],
                            outs[i].at[c, pl.ds(zbase, rpt)])

    return k(*rows_list, idx2d)


def _head_mats(heads, c):
    d = heads * c
    sp = np.zeros((d, 16), np.float32)
    st = np.zeros((16, d), np.float32)
    for h in range(heads):
        sp[h * c:(h + 1) * c, h] = 1.0
        st[h, h * c:(h + 1) * c] = 1.0
    return jnp.asarray(sp), jnp.asarray(st)


def _edge_stage(q, k, v, src_g, dst_g, dst_s, attr, we, heads, c):
    d = heads * c
    sp, st = _head_mats(heads, c)
    ks, qd, vs = _gather_qkv(k, q, v, src_g, dst_g)
    widths = (96, 96) if d > 96 else (d,)
    ea, *msgs = _alpha_msg(qd, ks, vs, attr, we, sp, st,
                           1.0 / float(np.sqrt(c)), widths)
    dparts, opart0 = _scatter_add_multi((ea, msgs[0]), dst_s, N_NODES)
    oparts = [opart0]
    if len(msgs) > 1:
        oparts += _scatter_add_multi(tuple(msgs[1:]), dst_s, N_NODES)
    return oparts, dparts, st


def kernel(x, edge_index, edge_attr, linA_W, linA_b, linB_W, linB_b,
           c1_Wq, c1_bq, c1_Wk, c1_bk, c1_Wv, c1_bv, c1_We, c1_Ws, c1_bs,
           c4_Wq, c4_bq, c4_Wk, c4_bk, c4_Wv, c4_bv, c4_We, c4_Ws, c4_bs,
           lin1_W, lin1_b, lin2_W, lin2_b):
    r1 = lambda b: b.reshape(1, -1)
    src_g = edge_index[0].reshape(-1, CHG)
    dst_g = edge_index[1].reshape(-1, CHG)
    dst_s = edge_index[1].reshape(-1, CH)

    h, q, k, v = _mlp_proj(x, linA_W, r1(linA_b), linB_W, r1(linB_b),
                           c1_Wq, r1(c1_bq), c1_Wk, r1(c1_bk),
                           c1_Wv, r1(c1_bv))
    oparts, dparts, st1 = _edge_stage(q, k, v, src_g, dst_g, dst_s,
                                      edge_attr, c1_We, 8, 24)
    h, q, k, v = _combine_proj(oparts, dparts, h, c1_Ws, r1(c1_bs), st1,
                               c4_Wq, r1(c4_bq), c4_Wk, r1(c4_bk),
                               c4_Wv, r1(c4_bv))
    oparts, dparts, st2 = _edge_stage(q, k, v, src_g, dst_g, dst_s,
                                      edge_attr, c4_We, 8, 8)
    return _combine_head(oparts, dparts, h, c4_Ws, r1(c4_bs), st2,
                         lin1_W, r1(lin1_b), lin2_W, r1(lin2_b))


# speedup vs baseline: 1.0234x; 1.0234x over previous
import functools

import jax
import jax.numpy as jnp
import numpy as np
from jax import lax
from jax.experimental import pallas as pl
from jax.experimental.pallas import tpu as pltpu
from jax.experimental.pallas import tpu_sc as plsc

N_NODES = 10000
N_EDGES = 160000
NC, NS = 2, 16
NW = NC * NS
ROW_BLK = 1000
EDGE_BLK = 4000
CH = 200
CHG = 40


def _elu(x):
    return jnp.where(x > 0, x, jnp.exp(jnp.minimum(x, 0.0)) - 1.0)


def _dot(a, b):
    return jnp.dot(a, b, preferred_element_type=jnp.float32)


def _full(a):
    return pl.BlockSpec(a.shape, lambda i: (0, 0))


def _mlp_proj_body(x_ref, wa_ref, ba_ref, wb_ref, bb_ref,
                   wq_ref, bq_ref, wk_ref, bk_ref, wv_ref, bv_ref,
                   h_ref, q_ref, k_ref, v_ref):
    h = _elu(_dot(x_ref[...], wa_ref[...]) + ba_ref[...])
    h = _elu(_dot(h, wb_ref[...]) + bb_ref[...])
    h_ref[...] = h
    q_ref[...] = _dot(h, wq_ref[...]) + bq_ref[...]
    k_ref[...] = _dot(h, wk_ref[...]) + bk_ref[...]
    v_ref[...] = _dot(h, wv_ref[...]) + bv_ref[...]


def _mlp_proj(x, wa, ba, wb, bb, wq, bq, wk, bk, wv, bv):
    n, din = x.shape
    dh = wb.shape[1]
    d = wq.shape[1]
    grid = n // ROW_BLK
    return pl.pallas_call(
        _mlp_proj_body,
        grid=(grid,),
        in_specs=[pl.BlockSpec((ROW_BLK, din), lambda i: (i, 0))]
                 + [_full(a) for a in (wa, ba, wb, bb, wq, bq, wk, bk,
                                       wv, bv)],
        out_specs=[pl.BlockSpec((ROW_BLK, dh), lambda i: (i, 0))]
                  + [pl.BlockSpec((ROW_BLK, d), lambda i: (i, 0))] * 3,
        out_shape=[jax.ShapeDtypeStruct((n, dh), jnp.float32)]
                  + [jax.ShapeDtypeStruct((n, d), jnp.float32)] * 3,
    )(x, wa, ba, wb, bb, wq, bq, wk, bk, wv, bv)


def _alpha_msg_body(qd_ref, ks_ref, vs_ref, attr_ref, we_ref, sp_ref,
                    st_ref, ea_ref, *o_refs, widths, scale):
    e = _dot(attr_ref[...], we_ref[...])
    logit = _dot(qd_ref[...] * (ks_ref[...] + e), sp_ref[...]) * scale
    ea = jnp.exp(logit)
    ea_ref[...] = ea
    m = (vs_ref[...] + e) * _dot(ea, st_ref[...])
    off = 0
    for o_ref, wd in zip(o_refs, widths):
        o_ref[...] = m[:, off:off + wd]
        off += wd


def _alpha_msg(qd, ks, vs, attr, we, sp, st, scale, widths):
    e, d = qd.shape
    grid = e // EDGE_BLK
    eb = lambda wd: pl.BlockSpec((EDGE_BLK, wd), lambda i: (i, 0))
    return pl.pallas_call(
        functools.partial(_alpha_msg_body, widths=tuple(widths),
                          scale=scale),
        grid=(grid,),
        in_specs=[eb(d), eb(d), eb(d), eb(7), _full(we), _full(sp),
                  _full(st)],
        out_specs=[eb(16)] + [eb(wd) for wd in widths],
        out_shape=[jax.ShapeDtypeStruct((e, 16), jnp.float32)]
                  + [jax.ShapeDtypeStruct((e, wd), jnp.float32)
                     for wd in widths],
    )(qd, ks, vs, attr, we, sp, st)


def _combine_block(refs, widths):
    np_ = len(widths)
    p_refs = refs[:np_]
    dp_ref, h_ref, ws_ref, bs_ref, st_ref = refs[np_:np_ + 5]
    rest = refs[np_ + 5:]
    base = _dot(h_ref[...], ws_ref[...]) + bs_ref[...]
    r = 1.0 / (dp_ref[0] + dp_ref[1] + 1e-16)
    rb = _dot(r, st_ref[...])
    cols = []
    off = 0
    for p_ref, wd in zip(p_refs, widths):
        cols.append(_elu((p_ref[0] + p_ref[1]) * rb[:, off:off + wd]
                         + base[:, off:off + wd]))
        off += wd
    o = cols[0] if len(cols) == 1 else jnp.concatenate(cols, axis=1)
    return o, rest


def _combine_proj_body(*refs, widths):
    o, rest = _combine_block(refs[:-4], widths)
    wq_ref, bq_ref, wk_ref, bk_ref, wv_ref, bv_ref = rest
    h_ref, q_ref, k_ref, v_ref = refs[-4:]
    h_ref[...] = o
    q_ref[...] = _dot(o, wq_ref[...]) + bq_ref[...]
    k_ref[...] = _dot(o, wk_ref[...]) + bk_ref[...]
    v_ref[...] = _dot(o, wv_ref[...]) + bv_ref[...]


def _combine_proj(parts, dparts, h, ws, bs, st, wq, bq, wk, bk, wv, bv):
    n, din = h.shape
    d = ws.shape[1]
    d2 = wq.shape[1]
    widths = tuple(p.shape[2] for p in parts)
    grid = n // ROW_BLK
    return pl.pallas_call(
        functools.partial(_combine_proj_body, widths=widths),
        grid=(grid,),
        in_specs=[pl.BlockSpec((2, ROW_BLK, wd), lambda i: (0, i, 0))
                  for wd in widths]
                 + [pl.BlockSpec((2, ROW_BLK, 16), lambda i: (0, i, 0)),
                    pl.BlockSpec((ROW_BLK, din), lambda i: (i, 0))]
                 + [_full(a) for a in (ws, bs, st, wq, bq, wk, bk, wv, bv)],
        out_specs=[pl.BlockSpec((ROW_BLK, d), lambda i: (i, 0))]
                  + [pl.BlockSpec((ROW_BLK, d2), lambda i: (i, 0))] * 3,
        out_shape=[jax.ShapeDtypeStruct((n, d), jnp.float32)]
                  + [jax.ShapeDtypeStruct((n, d2), jnp.float32)] * 3,
    )(*parts, dparts, h, ws, bs, st, wq, bq, wk, bk, wv, bv)


def _combine_head_body(*refs, widths):
    o, rest = _combine_block(refs[:-1], widths)
    w1_ref, b1_ref, w2_ref, b2_ref = rest
    y_ref = refs[-1]
    t = _elu(_dot(o, w1_ref[...]) + b1_ref[...])
    y_ref[...] = _dot(t, w2_ref[...]) + b2_ref[...]


def _combine_head(parts, dparts, h, ws, bs, st, w1, b1, w2, b2):
    n, din = h.shape
    widths = tuple(p.shape[2] for p in parts)
    grid = n // ROW_BLK
    return pl.pallas_call(
        functools.partial(_combine_head_body, widths=widths),
        grid=(grid,),
        in_specs=[pl.BlockSpec((2, ROW_BLK, wd), lambda i: (0, i, 0))
                  for wd in widths]
                 + [pl.BlockSpec((2, ROW_BLK, 16), lambda i: (0, i, 0)),
                    pl.BlockSpec((ROW_BLK, din), lambda i: (i, 0))]
                 + [_full(a) for a in (ws, bs, st, w1, b1, w2, b2)],
        out_specs=pl.BlockSpec((ROW_BLK, 1), lambda i: (i, 0)),
        out_shape=jax.ShapeDtypeStruct((n, 1), jnp.float32),
    )(*parts, dparts, h, ws, bs, st, w1, b1, w2, b2)


@functools.cache
def _sc_mesh():
    return plsc.VectorSubcoreMesh(core_axis_name="c", subcore_axis_name="s",
                                  num_cores=NC, num_subcores=NS)


def _gather_qkv(kt, qt, vt, src2d, dst2d):
    n_rows, d = kt.shape
    n_ch_tot, ch = src2d.shape
    e = n_ch_tot * ch
    per_w = e // NW
    n_ch = per_w // ch

    @functools.partial(
        pl.kernel, mesh=_sc_mesh(),
        out_type=[jax.ShapeDtypeStruct((e, d), jnp.float32)] * 3,
        compiler_params=pltpu.CompilerParams(use_tc_tiling_on_sc=False),
        scratch_types=[pltpu.VMEM((n_ch, ch), jnp.int32),
                       pltpu.VMEM((n_ch, ch), jnp.int32)]
                      + [pltpu.VMEM((ch, d), jnp.float32)] * 6
                      + [pltpu.SemaphoreType.DMA] * 12,
    )
    def k(kt_h, qt_h, vt_h, src_h, dst_h, ks_h, qd_h, vs_h,
          si_v, di_v, k0, k1, q0, q1, v0, v1, *sems):
        wid = lax.axis_index("s") * NC + lax.axis_index("c")
        cbase = wid * n_ch
        ebase = wid * per_w
        bufs = ((k0, k1), (q0, q1), (v0, v1))
        outs = (ks_h, qd_h, vs_h)
        tabs = (kt_h, qt_h, vt_h)
        idxs = (si_v, di_v, si_v)
        sg = (sems[0:2], sems[2:4], sems[4:6])
        sw = (sems[6:8], sems[8:10], sems[10:12])
        pltpu.sync_copy(src_h.at[pl.ds(cbase, n_ch)], si_v)
        pltpu.sync_copy(dst_h.at[pl.ds(cbase, n_ch)], di_v)

        def fire(g, b):
            for t in range(3):
                pltpu.async_copy(tabs[t].at[idxs[t].at[g]],
                                 bufs[t][b], sg[t][b])

        def wait_writes(g, b):
            for t in range(3):
                pltpu.make_async_copy(
                    bufs[t][b], outs[t].at[pl.ds(ebase + g * ch, ch)],
                    sw[t][b]).wait()

        def drain(g, b):
            for t in range(3):
                pltpu.make_async_copy(tabs[t].at[idxs[t].at[g]],
                                      bufs[t][b], sg[t][b]).wait()
                pltpu.async_copy(bufs[t][b],
                                 outs[t].at[pl.ds(ebase + g * ch, ch)],
                                 sw[t][b])

        fire(0, 0)

        def body(p, carry):
            for b in range(2):
                g = p * 2 + b
                nb = 1 - b

                @pl.when(jnp.logical_and(g + 1 < n_ch, g >= 1))
                def _():
                    wait_writes(g - 1, nb)

                @pl.when(g + 1 < n_ch)
                def _():
                    fire(g + 1, nb)

                drain(g, b)
            return carry

        lax.fori_loop(0, n_ch // 2, body, 0)
        if n_ch % 2:
            g = n_ch - 1
            drain(g, g % 2)
        wait_writes(n_ch - 2, (n_ch - 2) % 2)
        wait_writes(n_ch - 1, (n_ch - 1) % 2)

    return k(kt, qt, vt, src2d, dst2d)


def _scatter_add_multi(rows_list, idx2d, n_out):
    nr = len(rows_list)
    e = rows_list[0].shape[0]
    dims = [r.shape[1] for r in rows_list]
    n_ch_tot, ch = idx2d.shape
    per_w = e // NW
    n_ch = per_w // ch
    rpt = n_out // NS

    scratch = [pltpu.VMEM((n_ch, ch), jnp.int32)]
    for d in dims:
        scratch += [pltpu.VMEM((ch, d), jnp.float32)] * 2
    for d in dims:
        scratch.append(pltpu.VMEM_SHARED((n_out, d), jnp.float32))
    scratch += [pltpu.SemaphoreType.DMA] * (2 * nr)

    @functools.partial(
        pl.kernel, mesh=_sc_mesh(),
        out_type=[jax.ShapeDtypeStruct((NC, n_out, d), jnp.float32)
                  for d in dims],
        compiler_params=pltpu.CompilerParams(use_tc_tiling_on_sc=False),
        scratch_types=scratch,
    )
    def k(*args):
        rows_h = args[:nr]
        idx_h = args[nr]
        outs = args[nr + 1:2 * nr + 1]
        rest = args[2 * nr + 1:]
        idx_v = rest[0]
        bufs = [(rest[1 + 2 * i], rest[2 + 2 * i]) for i in range(nr)]
        accs = rest[1 + 2 * nr:1 + 3 * nr]
        sems = rest[1 + 3 * nr:]
        semp = [(sems[2 * i], sems[2 * i + 1]) for i in range(nr)]

        c = lax.axis_index("c")
        s = lax.axis_index("s")
        wid = s * NC + c
        zbase = s * rpt

        for i, d in enumerate(dims):
            b0 = bufs[i][0]

            def zero_buf(r, carry, b0=b0, d=d):
                for j in range(d // 16):
                    b0[r, pl.ds(j * 16, 16)] = jnp.zeros((16,),
                                                         jnp.float32)
                return carry

            lax.fori_loop(0, ch, zero_buf, 0)
            off = 0
            while off < rpt:
                step = min(ch, rpt - off)
                pltpu.sync_copy(b0.at[pl.ds(0, step)],
                                accs[i].at[pl.ds(zbase + off, step)])
                off += step

        cbase = wid * n_ch
        ebase = wid * per_w
        pltpu.sync_copy(idx_h.at[pl.ds(cbase, n_ch)], idx_v)
        for i in range(nr):
            pltpu.async_copy(rows_h[i].at[pl.ds(ebase, ch)],
                             bufs[i][0], semp[i][0])
        plsc.subcore_barrier()

        def body(p, carry):
            for b in range(2):
                g = p * 2 + b
                nb = 1 - b

                @pl.when(g + 1 < n_ch)
                def _():
                    for i in range(nr):
                        pltpu.async_copy(
                            rows_h[i].at[pl.ds(ebase + (g + 1) * ch, ch)],
                            bufs[i][nb], semp[i][nb])

                for i in range(nr):
                    pltpu.make_async_copy(
                        rows_h[i].at[pl.ds(ebase + g * ch, ch)],
                        bufs[i][b], semp[i][b]).wait()
                    pltpu.sync_copy(bufs[i][b], accs[i].at[idx_v.at[g]],
                                    add=True)
            return carry

        lax.fori_loop(0, n_ch // 2, body, 0)
        if n_ch % 2:
            g = n_ch - 1
            b = g % 2
            for i in range(nr):
                pltpu.make_async_copy(
                    rows_h[i].at[pl.ds(ebase + g * ch, ch)],
                    bufs[i][b], semp[i][b]).wait()
                pltpu.sync_copy(bufs[i][b], accs[i].at[idx_v.at[g]],
                                add=True)
        plsc.subcore_barrier()

        for i in range(nr):
            pltpu.sync_copy(accs[i].at[pl.ds(zbase, rpt)],
                            outs[i].at[c, pl.ds(zbase, rpt)])

    return k(*rows_list, idx2d)


def _head_mats(heads, c):
    d = heads * c
    sp = np.zeros((d, 16), np.float32)
    st = np.zeros((16, d), np.float32)
    for h in range(heads):
        sp[h * c:(h + 1) * c, h] = 1.0
        st[h, h * c:(h + 1) * c] = 1.0
    return jnp.asarray(sp), jnp.asarray(st)


def _edge_stage(q, k, v, src_g, dst_g, dst_s, attr, we, heads, c):
    d = heads * c
    sp, st = _head_mats(heads, c)
    ks, qd, vs = _gather_qkv(k, q, v, src_g, dst_g)
    widths = (96, 96) if d > 96 else (d,)
    ea, *msgs = _alpha_msg(qd, ks, vs, attr, we, sp, st,
                           1.0 / float(np.sqrt(c)), widths)
    dparts, = _scatter_add_multi((ea,), dst_s, N_NODES)
    oparts = [_scatter_add_multi((m,), dst_s, N_NODES)[0] for m in msgs]
    return oparts, dparts, st


def kernel(x, edge_index, edge_attr, linA_W, linA_b, linB_W, linB_b,
           c1_Wq, c1_bq, c1_Wk, c1_bk, c1_Wv, c1_bv, c1_We, c1_Ws, c1_bs,
           c4_Wq, c4_bq, c4_Wk, c4_bk, c4_Wv, c4_bv, c4_We, c4_Ws, c4_bs,
           lin1_W, lin1_b, lin2_W, lin2_b):
    r1 = lambda b: b.reshape(1, -1)
    src_g = edge_index[0].reshape(-1, CHG)
    dst_g = edge_index[1].reshape(-1, CHG)
    dst_s = edge_index[1].reshape(-1, CH)

    h, q, k, v = _mlp_proj(x, linA_W, r1(linA_b), linB_W, r1(linB_b),
                           c1_Wq, r1(c1_bq), c1_Wk, r1(c1_bk),
                           c1_Wv, r1(c1_bv))
    oparts, dparts, st1 = _edge_stage(q, k, v, src_g, dst_g, dst_s,
                                      edge_attr, c1_We, 8, 24)
    h, q, k, v = _combine_proj(oparts, dparts, h, c1_Ws, r1(c1_bs), st1,
                               c4_Wq, r1(c4_bq), c4_Wk, r1(c4_bk),
                               c4_Wv, r1(c4_bv))
    oparts, dparts, st2 = _edge_stage(q, k, v, src_g, dst_g, dst_s,
                                      edge_attr, c4_We, 8, 8)
    return _combine_head(oparts, dparts, h, c4_Ws, r1(c4_bs), st2,
                         lin1_W, r1(lin1_b), lin2_W, r1(lin2_b))


# speedup vs baseline: 1.0255x; 1.0021x over previous
import functools

import jax
import jax.numpy as jnp
import numpy as np
from jax import lax
from jax.experimental import pallas as pl
from jax.experimental.pallas import tpu as pltpu
from jax.experimental.pallas import tpu_sc as plsc

N_NODES = 10000
N_EDGES = 160000
NC, NS = 2, 16
NW = NC * NS
ROW_BLK = 2000
EDGE_BLK = 4000
CH = 200
CHG = 40


def _elu(x):
    return jnp.where(x > 0, x, jnp.exp(jnp.minimum(x, 0.0)) - 1.0)


def _dot(a, b):
    return jnp.dot(a, b, preferred_element_type=jnp.float32)


def _full(a):
    return pl.BlockSpec(a.shape, lambda i: (0, 0))


def _mlp_proj_body(x_ref, wa_ref, ba_ref, wb_ref, bb_ref,
                   wq_ref, bq_ref, wk_ref, bk_ref, wv_ref, bv_ref,
                   h_ref, q_ref, k_ref, v_ref):
    h = _elu(_dot(x_ref[...], wa_ref[...]) + ba_ref[...])
    h = _elu(_dot(h, wb_ref[...]) + bb_ref[...])
    h_ref[...] = h
    q_ref[...] = _dot(h, wq_ref[...]) + bq_ref[...]
    k_ref[...] = _dot(h, wk_ref[...]) + bk_ref[...]
    v_ref[...] = _dot(h, wv_ref[...]) + bv_ref[...]


def _mlp_proj(x, wa, ba, wb, bb, wq, bq, wk, bk, wv, bv):
    n, din = x.shape
    dh = wb.shape[1]
    d = wq.shape[1]
    grid = n // ROW_BLK
    return pl.pallas_call(
        _mlp_proj_body,
        grid=(grid,),
        in_specs=[pl.BlockSpec((ROW_BLK, din), lambda i: (i, 0))]
                 + [_full(a) for a in (wa, ba, wb, bb, wq, bq, wk, bk,
                                       wv, bv)],
        out_specs=[pl.BlockSpec((ROW_BLK, dh), lambda i: (i, 0))]
                  + [pl.BlockSpec((ROW_BLK, d), lambda i: (i, 0))] * 3,
        out_shape=[jax.ShapeDtypeStruct((n, dh), jnp.float32)]
                  + [jax.ShapeDtypeStruct((n, d), jnp.float32)] * 3,
    )(x, wa, ba, wb, bb, wq, bq, wk, bk, wv, bv)


def _alpha_msg_body(qd_ref, ks_ref, vs_ref, attr_ref, we_ref, sp_ref,
                    st_ref, ea_ref, *o_refs, widths, scale):
    e = _dot(attr_ref[...], we_ref[...])
    logit = _dot(qd_ref[...] * (ks_ref[...] + e), sp_ref[...]) * scale
    ea = jnp.exp(logit)
    ea_ref[...] = ea
    m = (vs_ref[...] + e) * _dot(ea, st_ref[...])
    off = 0
    for o_ref, wd in zip(o_refs, widths):
        o_ref[...] = m[:, off:off + wd]
        off += wd


def _alpha_msg(qd, ks, vs, attr, we, sp, st, scale, widths):
    e, d = qd.shape
    grid = e // EDGE_BLK
    eb = lambda wd: pl.BlockSpec((EDGE_BLK, wd), lambda i: (i, 0))
    return pl.pallas_call(
        functools.partial(_alpha_msg_body, widths=tuple(widths),
                          scale=scale),
        grid=(grid,),
        in_specs=[eb(d), eb(d), eb(d), eb(7), _full(we), _full(sp),
                  _full(st)],
        out_specs=[eb(16)] + [eb(wd) for wd in widths],
        out_shape=[jax.ShapeDtypeStruct((e, 16), jnp.float32)]
                  + [jax.ShapeDtypeStruct((e, wd), jnp.float32)
                     for wd in widths],
    )(qd, ks, vs, attr, we, sp, st)


def _combine_block(refs, widths):
    np_ = len(widths)
    p_refs = refs[:np_]
    dp_ref, h_ref, ws_ref, bs_ref, st_ref = refs[np_:np_ + 5]
    rest = refs[np_ + 5:]
    base = _dot(h_ref[...], ws_ref[...]) + bs_ref[...]
    r = 1.0 / (dp_ref[0] + dp_ref[1] + 1e-16)
    rb = _dot(r, st_ref[...])
    cols = []
    off = 0
    for p_ref, wd in zip(p_refs, widths):
        cols.append(_elu((p_ref[0] + p_ref[1]) * rb[:, off:off + wd]
                         + base[:, off:off + wd]))
        off += wd
    o = cols[0] if len(cols) == 1 else jnp.concatenate(cols, axis=1)
    return o, rest


def _combine_proj_body(*refs, widths):
    o, rest = _combine_block(refs[:-4], widths)
    wq_ref, bq_ref, wk_ref, bk_ref, wv_ref, bv_ref = rest
    h_ref, q_ref, k_ref, v_ref = refs[-4:]
    h_ref[...] = o
    q_ref[...] = _dot(o, wq_ref[...]) + bq_ref[...]
    k_ref[...] = _dot(o, wk_ref[...]) + bk_ref[...]
    v_ref[...] = _dot(o, wv_ref[...]) + bv_ref[...]


def _combine_proj(parts, dparts, h, ws, bs, st, wq, bq, wk, bk, wv, bv):
    n, din = h.shape
    d = ws.shape[1]
    d2 = wq.shape[1]
    widths = tuple(p.shape[2] for p in parts)
    grid = n // ROW_BLK
    return pl.pallas_call(
        functools.partial(_combine_proj_body, widths=widths),
        grid=(grid,),
        in_specs=[pl.BlockSpec((2, ROW_BLK, wd), lambda i: (0, i, 0))
                  for wd in widths]
                 + [pl.BlockSpec((2, ROW_BLK, 16), lambda i: (0, i, 0)),
                    pl.BlockSpec((ROW_BLK, din), lambda i: (i, 0))]
                 + [_full(a) for a in (ws, bs, st, wq, bq, wk, bk, wv, bv)],
        out_specs=[pl.BlockSpec((ROW_BLK, d), lambda i: (i, 0))]
                  + [pl.BlockSpec((ROW_BLK, d2), lambda i: (i, 0))] * 3,
        out_shape=[jax.ShapeDtypeStruct((n, d), jnp.float32)]
                  + [jax.ShapeDtypeStruct((n, d2), jnp.float32)] * 3,
    )(*parts, dparts, h, ws, bs, st, wq, bq, wk, bk, wv, bv)


def _combine_head_body(*refs, widths):
    o, rest = _combine_block(refs[:-1], widths)
    w1_ref, b1_ref, w2_ref, b2_ref = rest
    y_ref = refs[-1]
    t = _elu(_dot(o, w1_ref[...]) + b1_ref[...])
    y_ref[...] = _dot(t, w2_ref[...]) + b2_ref[...]


def _combine_head(parts, dparts, h, ws, bs, st, w1, b1, w2, b2):
    n, din = h.shape
    widths = tuple(p.shape[2] for p in parts)
    grid = n // ROW_BLK
    return pl.pallas_call(
        functools.partial(_combine_head_body, widths=widths),
        grid=(grid,),
        in_specs=[pl.BlockSpec((2, ROW_BLK, wd), lambda i: (0, i, 0))
                  for wd in widths]
                 + [pl.BlockSpec((2, ROW_BLK, 16), lambda i: (0, i, 0)),
                    pl.BlockSpec((ROW_BLK, din), lambda i: (i, 0))]
                 + [_full(a) for a in (ws, bs, st, w1, b1, w2, b2)],
        out_specs=pl.BlockSpec((ROW_BLK, 1), lambda i: (i, 0)),
        out_shape=jax.ShapeDtypeStruct((n, 1), jnp.float32),
    )(*parts, dparts, h, ws, bs, st, w1, b1, w2, b2)


@functools.cache
def _sc_mesh():
    return plsc.VectorSubcoreMesh(core_axis_name="c", subcore_axis_name="s",
                                  num_cores=NC, num_subcores=NS)


def _gather_qkv(kt, qt, vt, src2d, dst2d):
    n_rows, d = kt.shape
    n_ch_tot, ch = src2d.shape
    e = n_ch_tot * ch
    per_w = e // NW
    n_ch = per_w // ch

    @functools.partial(
        pl.kernel, mesh=_sc_mesh(),
        out_type=[jax.ShapeDtypeStruct((e, d), jnp.float32)] * 3,
        compiler_params=pltpu.CompilerParams(use_tc_tiling_on_sc=False),
        scratch_types=[pltpu.VMEM((n_ch, ch), jnp.int32),
                       pltpu.VMEM((n_ch, ch), jnp.int32)]
                      + [pltpu.VMEM((ch, d), jnp.float32)] * 6
                      + [pltpu.SemaphoreType.DMA] * 12,
    )
    def k(kt_h, qt_h, vt_h, src_h, dst_h, ks_h, qd_h, vs_h,
          si_v, di_v, k0, k1, q0, q1, v0, v1, *sems):
        wid = lax.axis_index("s") * NC + lax.axis_index("c")
        cbase = wid * n_ch
        ebase = wid * per_w
        bufs = ((k0, k1), (q0, q1), (v0, v1))
        outs = (ks_h, qd_h, vs_h)
        tabs = (kt_h, qt_h, vt_h)
        idxs = (si_v, di_v, si_v)
        sg = (sems[0:2], sems[2:4], sems[4:6])
        sw = (sems[6:8], sems[8:10], sems[10:12])
        pltpu.sync_copy(src_h.at[pl.ds(cbase, n_ch)], si_v)
        pltpu.sync_copy(dst_h.at[pl.ds(cbase, n_ch)], di_v)

        def fire(g, b):
            for t in range(3):
                pltpu.async_copy(tabs[t].at[idxs[t].at[g]],
                                 bufs[t][b], sg[t][b])

        def wait_writes(g, b):
            for t in range(3):
                pltpu.make_async_copy(
                    bufs[t][b], outs[t].at[pl.ds(ebase + g * ch, ch)],
                    sw[t][b]).wait()

        def drain(g, b):
            for t in range(3):
                pltpu.make_async_copy(tabs[t].at[idxs[t].at[g]],
                                      bufs[t][b], sg[t][b]).wait()
                pltpu.async_copy(bufs[t][b],
                                 outs[t].at[pl.ds(ebase + g * ch, ch)],
                                 sw[t][b])

        fire(0, 0)

        def body(p, carry):
            for b in range(2):
                g = p * 2 + b
                nb = 1 - b

                @pl.when(jnp.logical_and(g + 1 < n_ch, g >= 1))
                def _():
                    wait_writes(g - 1, nb)

                @pl.when(g + 1 < n_ch)
                def _():
                    fire(g + 1, nb)

                drain(g, b)
            return carry

        lax.fori_loop(0, n_ch // 2, body, 0)
        if n_ch % 2:
            g = n_ch - 1
            drain(g, g % 2)
        wait_writes(n_ch - 2, (n_ch - 2) % 2)
        wait_writes(n_ch - 1, (n_ch - 1) % 2)

    return k(kt, qt, vt, src2d, dst2d)


def _scatter_add_multi(rows_list, idx2d, n_out):
    nr = len(rows_list)
    e = rows_list[0].shape[0]
    dims = [r.shape[1] for r in rows_list]
    n_ch_tot, ch = idx2d.shape
    per_w = e // NW
    n_ch = per_w // ch
    rpt = n_out // NS

    scratch = [pltpu.VMEM((n_ch, ch), jnp.int32)]
    for d in dims:
        scratch += [pltpu.VMEM((ch, d), jnp.float32)] * 2
    for d in dims:
        scratch.append(pltpu.VMEM_SHARED((n_out, d), jnp.float32))
    scratch += [pltpu.SemaphoreType.DMA] * (2 * nr)

    @functools.partial(
        pl.kernel, mesh=_sc_mesh(),
        out_type=[jax.ShapeDtypeStruct((NC, n_out, d), jnp.float32)
                  for d in dims],
        compiler_params=pltpu.CompilerParams(use_tc_tiling_on_sc=False),
        scratch_types=scratch,
    )
    def k(*args):
        rows_h = args[:nr]
        idx_h = args[nr]
        outs = args[nr + 1:2 * nr + 1]
        rest = args[2 * nr + 1:]
        idx_v = rest[0]
        bufs = [(rest[1 + 2 * i], rest[2 + 2 * i]) for i in range(nr)]
        accs = rest[1 + 2 * nr:1 + 3 * nr]
        sems = rest[1 + 3 * nr:]
        semp = [(sems[2 * i], sems[2 * i + 1]) for i in range(nr)]

        c = lax.axis_index("c")
        s = lax.axis_index("s")
        wid = s * NC + c
        zbase = s * rpt

        for i, d in enumerate(dims):
            b0 = bufs[i][0]

            def zero_buf(r, carry, b0=b0, d=d):
                for j in range(d // 16):
                    b0[r, pl.ds(j * 16, 16)] = jnp.zeros((16,),
                                                         jnp.float32)
                return carry

            lax.fori_loop(0, ch, zero_buf, 0)
            off = 0
            while off < rpt:
                step = min(ch, rpt - off)
                pltpu.sync_copy(b0.at[pl.ds(0, step)],
                                accs[i].at[pl.ds(zbase + off, step)])
                off += step

        cbase = wid * n_ch
        ebase = wid * per_w
        pltpu.sync_copy(idx_h.at[pl.ds(cbase, n_ch)], idx_v)
        for i in range(nr):
            pltpu.async_copy(rows_h[i].at[pl.ds(ebase, ch)],
                             bufs[i][0], semp[i][0])
        plsc.subcore_barrier()

        def body(p, carry):
            for b in range(2):
                g = p * 2 + b
                nb = 1 - b

                @pl.when(g + 1 < n_ch)
                def _():
                    for i in range(nr):
                        pltpu.async_copy(
                            rows_h[i].at[pl.ds(ebase + (g + 1) * ch, ch)],
                            bufs[i][nb], semp[i][nb])

                for i in range(nr):
                    pltpu.make_async_copy(
                        rows_h[i].at[pl.ds(ebase + g * ch, ch)],
                        bufs[i][b], semp[i][b]).wait()
                    pltpu.sync_copy(bufs[i][b], accs[i].at[idx_v.at[g]],
                                    add=True)
            return carry

        lax.fori_loop(0, n_ch // 2, body, 0)
        if n_ch % 2:
            g = n_ch - 1
            b = g % 2
            for i in range(nr):
                pltpu.make_async_copy(
                    rows_h[i].at[pl.ds(ebase + g * ch, ch)],
                    bufs[i][b], semp[i][b]).wait()
                pltpu.sync_copy(bufs[i][b], accs[i].at[idx_v.at[g]],
                                add=True)
        plsc.subcore_barrier()

        for i in range(nr):
            pltpu.sync_copy(accs[i].at[pl.ds(zbase, rpt)],
                            outs[i].at[c, pl.ds(zbase, rpt)])

    return k(*rows_list, idx2d)


def _head_mats(heads, c):
    d = heads * c
    sp = np.zeros((d, 16), np.float32)
    st = np.zeros((16, d), np.float32)
    for h in range(heads):
        sp[h * c:(h + 1) * c, h] = 1.0
        st[h, h * c:(h + 1) * c] = 1.0
    return jnp.asarray(sp), jnp.asarray(st)


def _edge_stage(q, k, v, src_g, dst_g, dst_s, attr, we, heads, c):
    d = heads * c
    sp, st = _head_mats(heads, c)
    ks, qd, vs = _gather_qkv(k, q, v, src_g, dst_g)
    widths = (96, 96) if d > 96 else (d,)
    ea, *msgs = _alpha_msg(qd, ks, vs, attr, we, sp, st,
                           1.0 / float(np.sqrt(c)), widths)
    dparts, = _scatter_add_multi((ea,), dst_s, N_NODES)
    oparts = [_scatter_add_multi((m,), dst_s, N_NODES)[0] for m in msgs]
    return oparts, dparts, st


def kernel(x, edge_index, edge_attr, linA_W, linA_b, linB_W, linB_b,
           c1_Wq, c1_bq, c1_Wk, c1_bk, c1_Wv, c1_bv, c1_We, c1_Ws, c1_bs,
           c4_Wq, c4_bq, c4_Wk, c4_bk, c4_Wv, c4_bv, c4_We, c4_Ws, c4_bs,
           lin1_W, lin1_b, lin2_W, lin2_b):
    r1 = lambda b: b.reshape(1, -1)
    src_g = edge_index[0].reshape(-1, CHG)
    dst_g = edge_index[1].reshape(-1, CHG)
    dst_s = edge_index[1].reshape(-1, CH)

    h, q, k, v = _mlp_proj(x, linA_W, r1(linA_b), linB_W, r1(linB_b),
                           c1_Wq, r1(c1_bq), c1_Wk, r1(c1_bk),
                           c1_Wv, r1(c1_bv))
    oparts, dparts, st1 = _edge_stage(q, k, v, src_g, dst_g, dst_s,
                                      edge_attr, c1_We, 8, 24)
    h, q, k, v = _combine_proj(oparts, dparts, h, c1_Ws, r1(c1_bs), st1,
                               c4_Wq, r1(c4_bq), c4_Wk, r1(c4_bk),
                               c4_Wv, r1(c4_bv))
    oparts, dparts, st2 = _edge_stage(q, k, v, src_g, dst_g, dst_s,
                                      edge_attr, c4_We, 8, 8)
    return _combine_head(oparts, dparts, h, c4_Ws, r1(c4_bs), st2,
                         lin1_W, r1(lin1_b), lin2_W, r1(lin2_b))
